# baseline (speedup 1.0000x reference)
import functools

import jax
import jax.numpy as jnp
from jax import lax
from jax.experimental import pallas as pl
from jax.experimental.pallas import tpu as pltpu
from jax.experimental.pallas import tpu_sc as plsc

_B = 128
_S = 512
_D = 768
_NW = 32
_SBLK = _S // _NW
_CB = 4
_C = _CB * _SBLK
_NCHUNK = _B // _CB
_LANES = 16
_KD = _D // _LANES

_DNUMS = lax.GatherDimensionNumbers(
    offset_dims=(), collapsed_slice_dims=(0,), start_index_map=(0,))


def _permute(v, idx):
    return lax.gather(v, idx.reshape(_LANES, 1), _DNUMS, (1,),
                      mode=lax.GatherScatterMode.PROMISE_IN_BOUNDS)


def _allsum(v):
    lanes = lax.iota(jnp.int32, _LANES)
    for sh in (8, 4, 2, 1):
        v = v + _permute(v, lanes ^ sh)
    return v


def _rsqrt(x):
    i = lax.bitcast_convert_type(x, jnp.int32)
    i = 0x5F3759DF - lax.shift_right_arithmetic(i, 1)
    y = lax.bitcast_convert_type(i, jnp.float32)
    for _ in range(3):
        y = y * (1.5 - 0.5 * x * y * y)
    return y


def _body(x_hbm, seg_hbm, tok_hbm, segtab_hbm, pos_hbm, gamma_hbm, beta_hbm,
          out_hbm, idx_v, seg_v, rows_v, cache_v, delta_v, segtab_v, gamma_v,
          beta_v, gs0, gs1, os0, os1, ss0, ss1):
    wid = lax.axis_index("s") * 2 + lax.axis_index("c")
    s0 = wid * _SBLK
    gsem = (gs0, gs1)
    osem = (os0, os1)
    ssem = (ss0, ss1)

    pltpu.sync_copy(gamma_hbm, gamma_v)
    pltpu.sync_copy(beta_hbm, beta_v)
    pltpu.sync_copy(segtab_hbm, segtab_v)
    pltpu.sync_copy(pos_hbm.at[pl.ds(s0, _SBLK)], cache_v)

    for k in range(_KD):
        dsl = pl.ds(k * _LANES, _LANES)
        delta_v[dsl] = segtab_v[1, dsl] - segtab_v[0, dsl]

    def add_seg(jj, carry):
        for k in range(_KD):
            dsl = pl.ds(k * _LANES, _LANES)
            cache_v[jj, dsl] = cache_v[jj, dsl] + segtab_v[0, dsl]
        return carry

    lax.fori_loop(0, _SBLK, add_seg, 0)

    def stage(g, p, sync):
        b0 = g * _CB
        for u in range(_CB):
            off = (b0 + u) * _S + s0
            dst_i = idx_v.at[p, pl.ds(u * _SBLK, _SBLK)]
            dst_s = seg_v.at[p, pl.ds(u * _SBLK, _SBLK)]
            if sync:
                pltpu.sync_copy(x_hbm.at[pl.ds(off, _SBLK)], dst_i)
                pltpu.sync_copy(seg_hbm.at[pl.ds(off, _SBLK)], dst_s)
            else:
                pltpu.async_copy(x_hbm.at[pl.ds(off, _SBLK)], dst_i, ssem[p])
                pltpu.async_copy(seg_hbm.at[pl.ds(off, _SBLK)], dst_s,
                                 ssem[p])

    def wait_stage(p):
        for u in range(_CB):
            pltpu.make_async_copy(
                x_hbm.at[pl.ds(0, _SBLK)],
                idx_v.at[p, pl.ds(u * _SBLK, _SBLK)], ssem[p]).wait()
            pltpu.make_async_copy(
                seg_hbm.at[pl.ds(0, _SBLK)],
                seg_v.at[p, pl.ds(u * _SBLK, _SBLK)], ssem[p]).wait()

    def fire_gather(p):
        pltpu.async_copy(tok_hbm.at[idx_v.at[p]], rows_v.at[p], gsem[p])

    def wait_gather(p):
        pltpu.make_async_copy(tok_hbm.at[idx_v.at[p]], rows_v.at[p],
                              gsem[p]).wait()

    def fire_out(g, p):
        b0 = g * _CB
        for u in range(_CB):
            off = (b0 + u) * _S + s0
            pltpu.async_copy(rows_v.at[p, pl.ds(u * _SBLK, _SBLK)],
                             out_hbm.at[pl.ds(off, _SBLK)], osem[p])

    def wait_out(p):
        for u in range(_CB):
            pltpu.make_async_copy(rows_v.at[p, pl.ds(u * _SBLK, _SBLK)],
                                  out_hbm.at[pl.ds(0, _SBLK)], osem[p]).wait()

    def compute(p):
        rows = rows_v.at[p]
        segs = seg_v.at[p]

        def jj_body(jj, carry):
            segf = []
            for u in range(_CB):
                sve = segs[pl.ds(u * _SBLK, _SBLK)]
                sv = _permute(sve, jnp.broadcast_to(jj, (_LANES,)))
                segf.append(sv.astype(jnp.float32))
            ts = [u * _SBLK + jj for u in range(_CB)]
            acc = [jnp.zeros((_LANES,), jnp.float32) for _ in range(_CB)]
            acc2 = [jnp.zeros((_LANES,), jnp.float32) for _ in range(_CB)]
            for k in range(_KD):
                dsl = pl.ds(k * _LANES, _LANES)
                c = cache_v[jj, dsl]
                d = delta_v[dsl]
                for u in range(_CB):
                    v = rows[ts[u], dsl] + c + segf[u] * d
                    rows[ts[u], dsl] = v
                    acc[u] = acc[u] + v
                    acc2[u] = acc2[u] + v * v
            mean = []
            inv = []
            for u in range(_CB):
                m = _allsum(acc[u]) * (1.0 / _D)
                m2 = _allsum(acc2[u]) * (1.0 / _D)
                mean.append(m)
                inv.append(_rsqrt(m2 - m * m + 1e-5))
            for k in range(_KD):
                dsl = pl.ds(k * _LANES, _LANES)
                gmv = gamma_v[dsl]
                btv = beta_v[dsl]
                for u in range(_CB):
                    v = rows[ts[u], dsl]
                    rows[ts[u], dsl] = ((v - mean[u]) * inv[u] * gmv + btv)
            return carry

        lax.fori_loop(0, _SBLK, jj_body, 0)

    stage(0, 0, sync=True)
    fire_gather(0)
    stage(1, 1, sync=False)

    def outer(g2, carry):
        for p in range(2):
            g = g2 * 2 + p
            q = 1 - p
            wait_gather(p)

            @pl.when(g + 1 < _NCHUNK)
            def _():
                wait_stage(q)

                @pl.when(g >= 1)
                def _():
                    wait_out(q)

                fire_gather(q)

            compute(p)
            fire_out(g, p)

            @pl.when(g + 2 < _NCHUNK)
            def _():
                stage(g + 2, p, sync=False)

        return carry

    lax.fori_loop(0, _NCHUNK // 2, outer, 0)
    wait_out(0)
    wait_out(1)


@jax.jit
def _run(xf, sf, tok_table, seg_table, pos_table, gamma, beta):
    call = functools.partial(
        pl.kernel,
        out_type=jax.ShapeDtypeStruct((_B * _S, _D), jnp.float32),
        mesh=plsc.VectorSubcoreMesh(core_axis_name="c", subcore_axis_name="s"),
        scratch_types=[
            pltpu.VMEM((2, _C), jnp.int32),
            pltpu.VMEM((2, _C), jnp.int32),
            pltpu.VMEM((2, _C, _D), jnp.float32),
            pltpu.VMEM((_SBLK, _D), jnp.float32),
            pltpu.VMEM((_D,), jnp.float32),
            pltpu.VMEM((2, _D), jnp.float32),
            pltpu.VMEM((_D,), jnp.float32),
            pltpu.VMEM((_D,), jnp.float32),
            pltpu.SemaphoreType.DMA,
            pltpu.SemaphoreType.DMA,
            pltpu.SemaphoreType.DMA,
            pltpu.SemaphoreType.DMA,
            pltpu.SemaphoreType.DMA,
            pltpu.SemaphoreType.DMA,
        ],
    )(_body)
    return call(xf, sf, tok_table, seg_table, pos_table, gamma, beta)


def kernel(x, seg, tok_table, seg_table, pos_table, gamma, beta):
    xf = x.reshape(-1)
    sf = seg.reshape(-1)
    out = _run(xf, sf, tok_table, seg_table, pos_table, gamma, beta)
    return out.reshape(x.shape[0], x.shape[1], tok_table.shape[1])


# speedup vs baseline: 2.4031x; 2.4031x over previous
import functools

import jax
import jax.numpy as jnp
from jax import lax
from jax.experimental import pallas as pl
from jax.experimental.pallas import tpu as pltpu
from jax.experimental.pallas import tpu_sc as plsc

_B = 128
_S = 512
_D = 768
_NW = 32
_SBLK = _S // _NW
_CB = 2
_C = _CB * _SBLK
_NCHUNK = _B // _CB
_LANES = 16
_KD = _D // _LANES

_DNUMS = lax.GatherDimensionNumbers(
    offset_dims=(), collapsed_slice_dims=(0,), start_index_map=(0,))


def _permute(v, idx):
    return lax.gather(v, idx.reshape(_LANES, 1), _DNUMS, (1,),
                      mode=lax.GatherScatterMode.PROMISE_IN_BOUNDS)


def _allsum(v):
    lanes = lax.iota(jnp.int32, _LANES)
    for sh in (8, 4, 2, 1):
        v = v + _permute(v, lanes ^ sh)
    return v


def _rsqrt(x):
    i = lax.bitcast_convert_type(x, jnp.int32)
    i = 0x5F3759DF - lax.shift_right_arithmetic(i, 1)
    y = lax.bitcast_convert_type(i, jnp.float32)
    for _ in range(3):
        y = y * (1.5 - 0.5 * x * y * y)
    return y


def _body(x_hbm, seg_hbm, tok_hbm, segtab_hbm, pos_hbm, gamma_hbm, beta_hbm,
          out_hbm, idx_v, seg_v, rows_v, out_sv, cache_v, delta_v, segtab_v,
          gamma_v, beta_v, gs0, gs1, os0, os1, ss0, ss1):
    wid = lax.axis_index("s") * 2 + lax.axis_index("c")
    s0 = wid * _SBLK
    gsem = (gs0, gs1)
    osem = (os0, os1)
    ssem = (ss0, ss1)

    pltpu.sync_copy(gamma_hbm, gamma_v)
    pltpu.sync_copy(beta_hbm, beta_v)
    pltpu.sync_copy(segtab_hbm, segtab_v)
    pltpu.sync_copy(pos_hbm.at[pl.ds(s0, _SBLK)], cache_v)

    for k in range(_KD):
        dsl = pl.ds(k * _LANES, _LANES)
        delta_v[dsl] = segtab_v[1, dsl] - segtab_v[0, dsl]

    def add_seg(jj, carry):
        for k in range(_KD):
            dsl = pl.ds(k * _LANES, _LANES)
            cache_v[jj, dsl] = cache_v[jj, dsl] + segtab_v[0, dsl]
        return carry

    lax.fori_loop(0, _SBLK, add_seg, 0)

    def stage(g, p, sync):
        b0 = g * _CB
        for u in range(_CB):
            off = (b0 + u) * _S + s0
            dst_i = idx_v.at[p, pl.ds(u * _SBLK, _SBLK)]
            dst_s = seg_v.at[p, pl.ds(u * _SBLK, _SBLK)]
            if sync:
                pltpu.sync_copy(x_hbm.at[pl.ds(off, _SBLK)], dst_i)
                pltpu.sync_copy(seg_hbm.at[pl.ds(off, _SBLK)], dst_s)
            else:
                pltpu.async_copy(x_hbm.at[pl.ds(off, _SBLK)], dst_i, ssem[p])
                pltpu.async_copy(seg_hbm.at[pl.ds(off, _SBLK)], dst_s,
                                 ssem[p])

    def wait_stage(p):
        for u in range(_CB):
            pltpu.make_async_copy(
                x_hbm.at[pl.ds(0, _SBLK)],
                idx_v.at[p, pl.ds(u * _SBLK, _SBLK)], ssem[p]).wait()
            pltpu.make_async_copy(
                seg_hbm.at[pl.ds(0, _SBLK)],
                seg_v.at[p, pl.ds(u * _SBLK, _SBLK)], ssem[p]).wait()

    def fire_gather(p):
        pltpu.async_copy(tok_hbm.at[idx_v.at[p]], rows_v.at[p], gsem[p])

    def wait_gather(p):
        pltpu.make_async_copy(tok_hbm.at[idx_v.at[p]], rows_v.at[p],
                              gsem[p]).wait()

    def fire_out(g, p):
        b0 = g * _CB
        for u in range(_CB):
            off = (b0 + u) * _S + s0
            pltpu.async_copy(out_sv.at[p, pl.ds(u * _SBLK, _SBLK)],
                             out_hbm.at[pl.ds(off, _SBLK)], osem[p])

    def wait_out(p):
        for u in range(_CB):
            pltpu.make_async_copy(out_sv.at[p, pl.ds(u * _SBLK, _SBLK)],
                                  out_hbm.at[pl.ds(0, _SBLK)], osem[p]).wait()

    def compute(p):
        rows = rows_v.at[p]
        outs = out_sv.at[p]
        segs = seg_v.at[p]

        def jj_body(jj, carry):
            ts = [jj, _SBLK + jj]
            segf = []
            for w in range(2):
                sve = segs[pl.ds(w * _SBLK, _SBLK)]
                sv = _permute(sve, jnp.broadcast_to(jj, (_LANES,)))
                segf.append(sv.astype(jnp.float32))
            acc = [[jnp.zeros((_LANES,), jnp.float32) for _ in range(2)]
                   for _ in range(2)]
            acc2 = [[jnp.zeros((_LANES,), jnp.float32) for _ in range(2)]
                    for _ in range(2)]
            for k in range(_KD):
                dsl = pl.ds(k * _LANES, _LANES)
                c = cache_v[jj, dsl]
                d = delta_v[dsl]
                e = k & 1
                for w in range(2):
                    v = rows[ts[w], dsl] + c + segf[w] * d
                    acc[w][e] = acc[w][e] + v
                    acc2[w][e] = acc2[w][e] + v * v
            mean = []
            inv = []
            for w in range(2):
                m = _allsum(acc[w][0] + acc[w][1]) * (1.0 / _D)
                m2 = _allsum(acc2[w][0] + acc2[w][1]) * (1.0 / _D)
                mean.append(m)
                inv.append(_rsqrt(m2 - m * m + 1e-5))
            for k in range(_KD):
                dsl = pl.ds(k * _LANES, _LANES)
                c = cache_v[jj, dsl]
                d = delta_v[dsl]
                gmv = gamma_v[dsl]
                btv = beta_v[dsl]
                for w in range(2):
                    v = rows[ts[w], dsl] + c + segf[w] * d
                    outs[ts[w], dsl] = (v - mean[w]) * inv[w] * gmv + btv
            return carry

        lax.fori_loop(0, _SBLK, jj_body, 0)

    stage(0, 0, sync=True)
    fire_gather(0)
    stage(1, 1, sync=False)

    def outer(g2, carry):
        for p in range(2):
            g = g2 * 2 + p
            q = 1 - p
            wait_gather(p)

            @pl.when(g + 1 < _NCHUNK)
            def _():
                wait_stage(q)
                fire_gather(q)

            @pl.when(g >= 2)
            def _():
                wait_out(p)

            compute(p)
            fire_out(g, p)

            @pl.when(g + 2 < _NCHUNK)
            def _():
                stage(g + 2, p, sync=False)

        return carry

    lax.fori_loop(0, _NCHUNK // 2, outer, 0)
    wait_out(0)
    wait_out(1)


@jax.jit
def _run(xf, sf, tok_table, seg_table, pos_table, gamma, beta):
    call = functools.partial(
        pl.kernel,
        out_type=jax.ShapeDtypeStruct((_B * _S, _D), jnp.float32),
        mesh=plsc.VectorSubcoreMesh(core_axis_name="c", subcore_axis_name="s"),
        scratch_types=[
            pltpu.VMEM((2, _C), jnp.int32),
            pltpu.VMEM((2, _C), jnp.int32),
            pltpu.VMEM((2, _C, _D), jnp.float32),
            pltpu.VMEM((2, _C, _D), jnp.float32),
            pltpu.VMEM((_SBLK, _D), jnp.float32),
            pltpu.VMEM((_D,), jnp.float32),
            pltpu.VMEM((2, _D), jnp.float32),
            pltpu.VMEM((_D,), jnp.float32),
            pltpu.VMEM((_D,), jnp.float32),
            pltpu.SemaphoreType.DMA,
            pltpu.SemaphoreType.DMA,
            pltpu.SemaphoreType.DMA,
            pltpu.SemaphoreType.DMA,
            pltpu.SemaphoreType.DMA,
            pltpu.SemaphoreType.DMA,
        ],
    )(_body)
    return call(xf, sf, tok_table, seg_table, pos_table, gamma, beta)


def kernel(x, seg, tok_table, seg_table, pos_table, gamma, beta):
    xf = x.reshape(-1)
    sf = seg.reshape(-1)
    out = _run(xf, sf, tok_table, seg_table, pos_table, gamma, beta)
    return out.reshape(x.shape[0], x.shape[1], tok_table.shape[1])
